# baseline (speedup 1.0000x reference)
import functools

import jax
import jax.numpy as jnp
from jax import lax
from jax.experimental import pallas as pl
from jax.experimental.pallas import tpu as pltpu
from jax.experimental.pallas import tpu_sc as plsc

_N = 10000
_E = 160000
_EP = 163840
_ET = 5120
_NSUB = 40
_NS = 10112
_RT = 632
_W = 64
_MB = 400
_G = _N // _MB
_NEG = 0.2
_H = 4


@functools.lru_cache(maxsize=None)
def _make_sc_hop(C):
    mesh = plsc.VectorSubcoreMesh(core_axis_name="core", subcore_axis_name="sub",
                                  num_cores=2, num_subcores=16)

    @functools.partial(
        pl.kernel,
        out_type=jax.ShapeDtypeStruct((2, C, _NS, _W), jnp.float32),
        mesh=mesh,
        compiler_params=pltpu.CompilerParams(use_tc_tiling_on_sc=False),
        scratch_types=[
            pltpu.VMEM((_NSUB, 128), jnp.int32),
            pltpu.VMEM((_ET,), jnp.int32),
            pltpu.VMEM((128, _W), jnp.float32),
            pltpu.VMEM((128, _W), jnp.float32),
            pltpu.VMEM_SHARED((_NS, _W), jnp.float32),
            pltpu.VMEM_SHARED((_NS, _W), jnp.float32),
            pltpu.SemaphoreType.DMA,
            pltpu.SemaphoreType.DMA,
            pltpu.SemaphoreType.DMA,
            pltpu.SemaphoreType.DMA,
            pltpu.SemaphoreType.DMA,
        ],
    )
    def hop(x_hbm, src_hbm, dst_hbm, z_hbm, out_hbm,
            dst_v, idx_v, buf_a, buf_b, xbuf, acc,
            sem_ga, sem_gb, sem_sa, sem_sb, sem_f):
        cid = lax.axis_index("core")
        sid = lax.axis_index("sub")
        tid = cid * 16 + sid
        pltpu.sync_copy(src_hbm.at[pl.ds(tid * _ET, _ET)], idx_v)
        pltpu.sync_copy(dst_hbm.at[tid], dst_v)
        r0 = sid * _RT
        pltpu.sync_copy(z_hbm, acc.at[pl.ds(r0, _RT)])
        pltpu.sync_copy(x_hbm.at[0, pl.ds(r0, _RT)], xbuf.at[pl.ds(r0, _RT)])
        plsc.subcore_barrier()

        def chunk_body(c, carry):
            pltpu.async_copy(xbuf.at[idx_v.at[pl.ds(0, 128)]], buf_a, sem_ga)

            def pipe(j2, c2):
                for b in range(2):
                    j = j2 * 2 + b
                    if b == 0:
                        buf, sg, ss = buf_a, sem_ga, sem_sa
                        obuf, sog, sos = buf_b, sem_gb, sem_sb
                    else:
                        buf, sg, ss = buf_b, sem_gb, sem_sb
                        obuf, sog, sos = buf_a, sem_ga, sem_sa
                    pltpu.make_async_copy(
                        xbuf.at[idx_v.at[pl.ds(0, 128)]], buf, sg).wait()
                    pltpu.async_copy(buf, acc.at[dst_v.at[j]], ss, add=True)
                    @pl.when(j + 1 < _NSUB)
                    def _():
                        @pl.when(j >= 1)
                        def _():
                            pltpu.make_async_copy(
                                obuf, acc.at[dst_v.at[0]], sos).wait()
                        pltpu.async_copy(
                            xbuf.at[idx_v.at[pl.ds((j + 1) * 128, 128)]],
                            obuf, sog)
                return c2
            lax.fori_loop(0, _NSUB // 2, pipe, 0)
            pltpu.make_async_copy(buf_b, acc.at[dst_v.at[0]], sem_sb).wait()
            plsc.subcore_barrier()

            pltpu.async_copy(acc.at[pl.ds(r0, _RT)],
                             out_hbm.at[cid, c, pl.ds(r0, _RT)], sem_f)
            @pl.when(c + 1 < C)
            def _():
                pltpu.sync_copy(x_hbm.at[c + 1, pl.ds(r0, _RT)],
                                xbuf.at[pl.ds(r0, _RT)])
            pltpu.make_async_copy(acc.at[pl.ds(r0, _RT)],
                                  out_hbm.at[cid, c, pl.ds(r0, _RT)],
                                  sem_f).wait()
            pltpu.sync_copy(z_hbm, acc.at[pl.ds(r0, _RT)])
            plsc.subcore_barrier()
            return carry

        lax.fori_loop(0, C, chunk_body, 0)

    return hop


@functools.lru_cache(maxsize=None)
def _make_sc_degree():
    mesh = plsc.VectorSubcoreMesh(core_axis_name="core", subcore_axis_name="sub",
                                  num_cores=2, num_subcores=16)

    @functools.partial(
        pl.kernel,
        out_type=jax.ShapeDtypeStruct((2, _NS, 16), jnp.float32),
        mesh=mesh,
        compiler_params=pltpu.CompilerParams(use_tc_tiling_on_sc=False),
        scratch_types=[
            pltpu.VMEM((_NSUB, 128), jnp.int32),
            pltpu.VMEM((128, 16), jnp.float32),
            pltpu.VMEM_SHARED((_NS, 16), jnp.float32),
            pltpu.SemaphoreType.DMA,
        ],
    )
    def degree(dst_hbm, ones_hbm, z_hbm, out_hbm, dst_v, ones_v, acc, sem_s):
        cid = lax.axis_index("core")
        sid = lax.axis_index("sub")
        tid = cid * 16 + sid
        pltpu.sync_copy(dst_hbm.at[tid], dst_v)
        pltpu.sync_copy(ones_hbm, ones_v)
        r0 = sid * _RT
        pltpu.sync_copy(z_hbm, acc.at[pl.ds(r0, _RT)])
        plsc.subcore_barrier()

        def jb(j, c2):
            pltpu.async_copy(ones_v, acc.at[dst_v.at[j]], sem_s, add=True)
            return c2
        lax.fori_loop(0, _NSUB, jb, 0)

        def drain(j, c2):
            pltpu.make_async_copy(ones_v, acc.at[dst_v.at[0]], sem_s).wait()
            return c2
        lax.fori_loop(0, _NSUB, drain, 0)
        plsc.subcore_barrier()
        pltpu.sync_copy(acc.at[pl.ds(r0, _RT)], out_hbm.at[cid, pl.ds(r0, _RT)])

    return degree


def _mm(x, w, norm=None):
    m, kin = x.shape
    dout = w.shape[1]
    want_y = norm is not None
    in_specs = [
        pl.BlockSpec((_MB, kin), lambda i: (i, 0)),
        pl.BlockSpec((kin, dout), lambda i: (0, 0)),
    ]
    ins = [x, w]
    if want_y:
        in_specs.append(pl.BlockSpec((_MB, 128), lambda i: (i, 0)))
        ins.append(norm)

        C = dout // _W

        def body(x_ref, w_ref, n_ref, f_ref, y_ref):
            f = jnp.dot(x_ref[...], w_ref[...],
                        preferred_element_type=jnp.float32)
            f_ref[...] = f
            y = f * n_ref[...][:, 0:1]
            for c in range(C):
                y_ref[c] = y[:, c * _W:(c + 1) * _W]

        out_shape = (jax.ShapeDtypeStruct((m, dout), jnp.float32),
                     jax.ShapeDtypeStruct((C, _NS, _W), jnp.float32))
        out_specs = (pl.BlockSpec((_MB, dout), lambda i: (i, 0)),
                     pl.BlockSpec((C, _MB, _W), lambda i: (0, i, 0)))
    else:
        def body(x_ref, w_ref, f_ref):
            f_ref[...] = jnp.dot(x_ref[...], w_ref[...],
                                 preferred_element_type=jnp.float32)

        out_shape = jax.ShapeDtypeStruct((m, dout), jnp.float32)
        out_specs = pl.BlockSpec((_MB, dout), lambda i: (i, 0))
    return pl.pallas_call(
        body, grid=(_G,), in_specs=in_specs, out_specs=out_specs,
        out_shape=out_shape)(*ins)


def _hop_post(part, norm, want_y):
    C = part.shape[1]
    d = C * _W

    def mk_f(p_ref, n):
        cols = [(p_ref[0, c] + p_ref[1, c]) for c in range(C)]
        return jnp.concatenate(cols, axis=1) * n

    if want_y:
        def body(p_ref, n_ref, f_ref, y_ref):
            n = n_ref[...][:, 0:1]
            f = mk_f(p_ref, n)
            f_ref[...] = f
            y = f * n
            for c in range(C):
                y_ref[c] = y[:, c * _W:(c + 1) * _W]

        out_shape = (jax.ShapeDtypeStruct((_N, d), jnp.float32),
                     jax.ShapeDtypeStruct((C, _NS, _W), jnp.float32))
        out_specs = (pl.BlockSpec((_MB, d), lambda i: (i, 0)),
                     pl.BlockSpec((C, _MB, _W), lambda i: (0, i, 0)))
    else:
        def body(p_ref, n_ref, f_ref):
            n = n_ref[...][:, 0:1]
            f_ref[...] = mk_f(p_ref, n)

        out_shape = jax.ShapeDtypeStruct((_N, d), jnp.float32)
        out_specs = pl.BlockSpec((_MB, d), lambda i: (i, 0))
    return pl.pallas_call(
        body, grid=(_G,),
        in_specs=[pl.BlockSpec((2, C, _MB, _W), lambda i: (0, 0, i, 0)),
                  pl.BlockSpec((_MB, 128), lambda i: (i, 0))],
        out_specs=out_specs, out_shape=out_shape)(part, norm)


def _scale(x, norm):
    d = x.shape[1]
    C = d // _W

    def body(x_ref, n_ref, y_ref):
        y = x_ref[...] * n_ref[...][:, 0:1]
        for c in range(C):
            y_ref[c] = y[:, c * _W:(c + 1) * _W]

    return pl.pallas_call(
        body, grid=(_G,),
        in_specs=[pl.BlockSpec((_MB, d), lambda i: (i, 0)),
                  pl.BlockSpec((_MB, 128), lambda i: (i, 0))],
        out_specs=pl.BlockSpec((C, _MB, _W), lambda i: (0, i, 0)),
        out_shape=jax.ShapeDtypeStruct((C, _NS, _W), jnp.float32))(x, norm)


def _norm_from_deg(deg_part):
    def body(p_ref, o_ref):
        d = p_ref[0][:, 0:1] + p_ref[1][:, 0:1]
        n = lax.rsqrt(jnp.maximum(d, 1.0))
        o_ref[...] = jnp.broadcast_to(n, (_MB, 128))

    return pl.pallas_call(
        body, grid=(_G,),
        in_specs=[pl.BlockSpec((2, _MB, 16), lambda i: (0, i, 0))],
        out_specs=pl.BlockSpec((_MB, 128), lambda i: (i, 0)),
        out_shape=jax.ShapeDtypeStruct((_N, 128), jnp.float32))(deg_part)


def _attention_weights(fs, al_v, ar_v, oh):
    rl = fs[0] * al_v
    ra = [f * ar_v for f in fs]
    weights = []
    for hh in range(_H):
        sl = slice(hh * oh, (hh + 1) * oh)
        a_l = jnp.sum(rl[:, sl], axis=1, keepdims=True)
        logits = [a_l + jnp.sum(r[:, sl], axis=1, keepdims=True) for r in ra]
        logits = [jnp.where(t >= 0, t, _NEG * t) for t in logits]
        mx = jnp.maximum(jnp.maximum(logits[0], logits[1]),
                         jnp.maximum(logits[2], logits[3]))
        es = [jnp.exp(t - mx) for t in logits]
        inv = 1.0 / (es[0] + es[1] + es[2] + es[3])
        weights.append([e * inv for e in es])
    return weights


def _combine(fs, lin, al_v, ar_v):
    d = lin.shape[1]
    oh = d // _H

    def body(f0, f1, f2, f3, l_ref, al_ref, ar_ref, o_ref, s_ref, q_ref):
        i = pl.program_id(0)
        fs_v = [f0[...], f1[...], f2[...], f3[...]]
        lin_v = l_ref[...]
        wts = _attention_weights(fs_v, al_ref[...], ar_ref[...], oh)
        cols = []
        for hh in range(_H):
            sl = slice(hh * oh, (hh + 1) * oh)
            acc = lin_v[:, sl]
            for k in range(4):
                acc = acc + fs_v[k][:, sl] * wts[hh][k]
            cols.append(acc)
        out = jnp.concatenate(cols, axis=1)
        o_ref[...] = out

        @pl.when(i == 0)
        def _():
            s_ref[...] = jnp.zeros_like(s_ref)
            q_ref[...] = jnp.zeros_like(q_ref)

        s_ref[...] += jnp.sum(out, axis=0, keepdims=True)
        q_ref[...] += jnp.sum(out * out, axis=0, keepdims=True)

    blk = pl.BlockSpec((_MB, d), lambda i: (i, 0))
    vec = pl.BlockSpec((1, d), lambda i: (0, 0))
    return pl.pallas_call(
        body, grid=(_G,),
        in_specs=[blk, blk, blk, blk, blk, vec, vec],
        out_specs=(blk, vec, vec),
        out_shape=(jax.ShapeDtypeStruct((_N, d), jnp.float32),
                   jax.ShapeDtypeStruct((1, d), jnp.float32),
                   jax.ShapeDtypeStruct((1, d), jnp.float32)),
    )(*fs, lin, al_v, ar_v)


def _combine_last(fs, lin, al_v, ar_v, bias):
    d = lin.shape[1]
    oh = d // _H

    def body(f0, f1, f2, f3, l_ref, al_ref, ar_ref, b_ref, o_ref):
        fs_v = [f0[...], f1[...], f2[...], f3[...]]
        lin_v = l_ref[...]
        wts = _attention_weights(fs_v, al_ref[...], ar_ref[...], oh)
        total = None
        for hh in range(_H):
            sl = slice(hh * oh, (hh + 1) * oh)
            acc = lin_v[:, sl]
            for k in range(4):
                acc = acc + fs_v[k][:, sl] * wts[hh][k]
            total = acc if total is None else total + acc
        o_ref[...] = total * (1.0 / _H) + b_ref[...]

    blk = pl.BlockSpec((_MB, d), lambda i: (i, 0))
    vec = pl.BlockSpec((1, d), lambda i: (0, 0))
    return pl.pallas_call(
        body, grid=(_G,),
        in_specs=[blk, blk, blk, blk, blk, vec, vec,
                  pl.BlockSpec((1, oh), lambda i: (0, 0))],
        out_specs=pl.BlockSpec((_MB, oh), lambda i: (i, 0)),
        out_shape=jax.ShapeDtypeStruct((_N, oh), jnp.float32),
    )(*fs, lin, al_v, ar_v, bias)


def _bn_relu(x, sums, sumsq, g, b):
    d = x.shape[1]

    def body(x_ref, s_ref, q_ref, g_ref, b_ref, o_ref):
        mu = s_ref[...] * (1.0 / _N)
        var = q_ref[...] * (1.0 / _N) - mu * mu
        rstd = lax.rsqrt(var + 1e-5)
        y = (x_ref[...] - mu) * (rstd * g_ref[...]) + b_ref[...]
        o_ref[...] = jnp.maximum(y, 0.0)

    vec = pl.BlockSpec((1, d), lambda i: (0, 0))
    return pl.pallas_call(
        body, grid=(_G,),
        in_specs=[pl.BlockSpec((_MB, d), lambda i: (i, 0)), vec, vec, vec, vec],
        out_specs=pl.BlockSpec((_MB, d), lambda i: (i, 0)),
        out_shape=jax.ShapeDtypeStruct((_N, d), jnp.float32),
    )(x, sums, sumsq, g, b)


def kernel(feat, edge_index, Wfc0, al0, ar0, Wlin0, g0, b0,
           Wfc1, al1, ar1, Wlin1, g1, b1, Wfc2, al2, ar2, Wlin2, bias_last):
    src = edge_index[0]
    dst = edge_index[1]
    pad = _EP - _E
    src_p = jnp.concatenate([src, jnp.zeros((pad,), jnp.int32)])
    dst_p = jnp.concatenate([dst, jnp.full((pad,), _N, jnp.int32)])
    dst3 = dst_p.reshape(32, _NSUB, 128)
    zeros_w = jnp.zeros((_RT, _W), jnp.float32)
    zeros_16 = jnp.zeros((_RT, 16), jnp.float32)
    ones_16 = jnp.ones((128, 16), jnp.float32)

    deg_part = _make_sc_degree()(dst3, ones_16, zeros_16)
    norm = _norm_from_deg(deg_part)

    def propagate(y):
        hop = _make_sc_hop(y.shape[0])
        return hop(y, src_p, dst3, zeros_w)

    hs = [feat]
    y = _scale(feat, norm)
    for k in range(3):
        part = propagate(y)
        if k < 2:
            h_k, y = _hop_post(part, norm, True)
        else:
            h_k = _hop_post(part, norm, False)
        hs.append(h_k)
    fs = [_mm(h_k, Wfc0) for h_k in hs]
    lin = _mm(feat, Wlin0)
    out, s, q = _combine(fs, lin, al0.reshape(1, -1), ar0.reshape(1, -1))
    h = _bn_relu(out, s, q, g0.reshape(1, -1), b0.reshape(1, -1))

    f0, y = _mm(h, Wfc1, norm)
    lin = _mm(h, Wlin1)
    fs = [f0]
    for k in range(3):
        part = propagate(y)
        if k < 2:
            f_k, y = _hop_post(part, norm, True)
        else:
            f_k = _hop_post(part, norm, False)
        fs.append(f_k)
    out, s, q = _combine(fs, lin, al1.reshape(1, -1), ar1.reshape(1, -1))
    h = _bn_relu(out, s, q, g1.reshape(1, -1), b1.reshape(1, -1))

    f0, y = _mm(h, Wfc2, norm)
    lin = _mm(h, Wlin2)
    fs = [f0]
    for k in range(3):
        part = propagate(y)
        if k < 2:
            f_k, y = _hop_post(part, norm, True)
        else:
            f_k = _hop_post(part, norm, False)
        fs.append(f_k)
    return _combine_last(fs, lin, al2.reshape(1, -1), ar2.reshape(1, -1),
                         bias_last.reshape(1, -1))


# speedup vs baseline: 1.1589x; 1.1589x over previous
import functools

import jax
import jax.numpy as jnp
from jax import lax
from jax.experimental import pallas as pl
from jax.experimental.pallas import tpu as pltpu
from jax.experimental.pallas import tpu_sc as plsc

_N = 10000
_E = 160000
_EP = 163840
_ET = 5120
_ET2 = 10240
_NSUB2 = 80
_NSUB = 40
_NS = 10112
_RT = 632
_W = 64
_MB = 400
_G = _N // _MB
_NEG = 0.2
_H = 4


@functools.lru_cache(maxsize=None)
def _make_sc_hop(C):
    CL = C // 2
    mesh = plsc.VectorSubcoreMesh(core_axis_name="core", subcore_axis_name="sub",
                                  num_cores=2, num_subcores=16)

    @functools.partial(
        pl.kernel,
        out_type=jax.ShapeDtypeStruct((C, _NS, _W), jnp.float32),
        mesh=mesh,
        compiler_params=pltpu.CompilerParams(use_tc_tiling_on_sc=False),
        scratch_types=[
            pltpu.VMEM((_NSUB2, 128), jnp.int32),
            pltpu.VMEM((_ET2,), jnp.int32),
            pltpu.VMEM((128, _W), jnp.float32),
            pltpu.VMEM((128, _W), jnp.float32),
            pltpu.VMEM_SHARED((_NS, _W), jnp.float32),
            pltpu.VMEM_SHARED((_NS, _W), jnp.float32),
            pltpu.SemaphoreType.DMA,
            pltpu.SemaphoreType.DMA,
            pltpu.SemaphoreType.DMA,
            pltpu.SemaphoreType.DMA,
            pltpu.SemaphoreType.DMA,
        ],
    )
    def hop(x_hbm, src_hbm, dst_hbm, z_hbm, out_hbm,
            dst_v, idx_v, buf_a, buf_b, xbuf, acc,
            sem_ga, sem_gb, sem_sa, sem_sb, sem_f):
        cid = lax.axis_index("core")
        sid = lax.axis_index("sub")
        pltpu.sync_copy(src_hbm.at[pl.ds(sid * _ET2, _ET2)], idx_v)
        pltpu.sync_copy(dst_hbm.at[sid], dst_v)
        r0 = sid * _RT
        c0 = cid * CL
        pltpu.sync_copy(z_hbm, acc.at[pl.ds(r0, _RT)])
        pltpu.sync_copy(x_hbm.at[c0, pl.ds(r0, _RT)], xbuf.at[pl.ds(r0, _RT)])
        plsc.subcore_barrier()

        def chunk_body(cl, carry):
            c = c0 + cl
            pltpu.async_copy(xbuf.at[idx_v.at[pl.ds(0, 128)]], buf_a, sem_ga)

            def pipe(j2, c2):
                for b in range(2):
                    j = j2 * 2 + b
                    if b == 0:
                        buf, sg, ss = buf_a, sem_ga, sem_sa
                        obuf, sog, sos = buf_b, sem_gb, sem_sb
                    else:
                        buf, sg, ss = buf_b, sem_gb, sem_sb
                        obuf, sog, sos = buf_a, sem_ga, sem_sa
                    pltpu.make_async_copy(
                        xbuf.at[idx_v.at[pl.ds(0, 128)]], buf, sg).wait()
                    pltpu.async_copy(buf, acc.at[dst_v.at[j]], ss, add=True)
                    @pl.when(j + 1 < _NSUB2)
                    def _():
                        @pl.when(j >= 1)
                        def _():
                            pltpu.make_async_copy(
                                obuf, acc.at[dst_v.at[0]], sos).wait()
                        pltpu.async_copy(
                            xbuf.at[idx_v.at[pl.ds((j + 1) * 128, 128)]],
                            obuf, sog)
                return c2
            lax.fori_loop(0, _NSUB2 // 2, pipe, 0)
            pltpu.make_async_copy(buf_b, acc.at[dst_v.at[0]], sem_sb).wait()
            plsc.subcore_barrier()

            pltpu.async_copy(acc.at[pl.ds(r0, _RT)],
                             out_hbm.at[c, pl.ds(r0, _RT)], sem_f)
            @pl.when(cl + 1 < CL)
            def _():
                pltpu.sync_copy(x_hbm.at[c + 1, pl.ds(r0, _RT)],
                                xbuf.at[pl.ds(r0, _RT)])
            pltpu.make_async_copy(acc.at[pl.ds(r0, _RT)],
                                  out_hbm.at[c, pl.ds(r0, _RT)], sem_f).wait()
            pltpu.sync_copy(z_hbm, acc.at[pl.ds(r0, _RT)])
            plsc.subcore_barrier()
            return carry

        lax.fori_loop(0, CL, chunk_body, 0)

    return hop


@functools.lru_cache(maxsize=None)
def _make_sc_degree():
    mesh = plsc.VectorSubcoreMesh(core_axis_name="core", subcore_axis_name="sub",
                                  num_cores=2, num_subcores=16)

    @functools.partial(
        pl.kernel,
        out_type=jax.ShapeDtypeStruct((2, _NS, 16), jnp.float32),
        mesh=mesh,
        compiler_params=pltpu.CompilerParams(use_tc_tiling_on_sc=False),
        scratch_types=[
            pltpu.VMEM((_NSUB, 128), jnp.int32),
            pltpu.VMEM((128, 16), jnp.float32),
            pltpu.VMEM_SHARED((_NS, 16), jnp.float32),
            pltpu.SemaphoreType.DMA,
        ],
    )
    def degree(dst_hbm, ones_hbm, z_hbm, out_hbm, dst_v, ones_v, acc, sem_s):
        cid = lax.axis_index("core")
        sid = lax.axis_index("sub")
        tid = cid * 16 + sid
        pltpu.sync_copy(dst_hbm.at[tid], dst_v)
        pltpu.sync_copy(ones_hbm, ones_v)
        r0 = sid * _RT
        pltpu.sync_copy(z_hbm, acc.at[pl.ds(r0, _RT)])
        plsc.subcore_barrier()

        def jb(j, c2):
            pltpu.async_copy(ones_v, acc.at[dst_v.at[j]], sem_s, add=True)
            return c2
        lax.fori_loop(0, _NSUB, jb, 0)

        def drain(j, c2):
            pltpu.make_async_copy(ones_v, acc.at[dst_v.at[0]], sem_s).wait()
            return c2
        lax.fori_loop(0, _NSUB, drain, 0)
        plsc.subcore_barrier()
        pltpu.sync_copy(acc.at[pl.ds(r0, _RT)], out_hbm.at[cid, pl.ds(r0, _RT)])

    return degree


def _mm(x, w, norm=None):
    m, kin = x.shape
    dout = w.shape[1]
    want_y = norm is not None
    in_specs = [
        pl.BlockSpec((_MB, kin), lambda i: (i, 0)),
        pl.BlockSpec((kin, dout), lambda i: (0, 0)),
    ]
    ins = [x, w]
    if want_y:
        in_specs.append(pl.BlockSpec((_MB, 128), lambda i: (i, 0)))
        ins.append(norm)

        C = dout // _W

        def body(x_ref, w_ref, n_ref, f_ref, y_ref):
            f = jnp.dot(x_ref[...], w_ref[...],
                        preferred_element_type=jnp.float32)
            f_ref[...] = f
            y = f * n_ref[...][:, 0:1]
            for c in range(C):
                y_ref[c] = y[:, c * _W:(c + 1) * _W]

        out_shape = (jax.ShapeDtypeStruct((m, dout), jnp.float32),
                     jax.ShapeDtypeStruct((C, _NS, _W), jnp.float32))
        out_specs = (pl.BlockSpec((_MB, dout), lambda i: (i, 0)),
                     pl.BlockSpec((C, _MB, _W), lambda i: (0, i, 0)))
    else:
        def body(x_ref, w_ref, f_ref):
            f_ref[...] = jnp.dot(x_ref[...], w_ref[...],
                                 preferred_element_type=jnp.float32)

        out_shape = jax.ShapeDtypeStruct((m, dout), jnp.float32)
        out_specs = pl.BlockSpec((_MB, dout), lambda i: (i, 0))
    return pl.pallas_call(
        body, grid=(_G,), in_specs=in_specs, out_specs=out_specs,
        out_shape=out_shape)(*ins)


def _hop_post(part, norm, want_y):
    C = part.shape[0]
    d = C * _W

    def mk_f(p_ref, n):
        cols = [p_ref[c] for c in range(C)]
        return jnp.concatenate(cols, axis=1) * n

    if want_y:
        def body(p_ref, n_ref, f_ref, y_ref):
            n = n_ref[...][:, 0:1]
            f = mk_f(p_ref, n)
            f_ref[...] = f
            y = f * n
            for c in range(C):
                y_ref[c] = y[:, c * _W:(c + 1) * _W]

        out_shape = (jax.ShapeDtypeStruct((_N, d), jnp.float32),
                     jax.ShapeDtypeStruct((C, _NS, _W), jnp.float32))
        out_specs = (pl.BlockSpec((_MB, d), lambda i: (i, 0)),
                     pl.BlockSpec((C, _MB, _W), lambda i: (0, i, 0)))
    else:
        def body(p_ref, n_ref, f_ref):
            n = n_ref[...][:, 0:1]
            f_ref[...] = mk_f(p_ref, n)

        out_shape = jax.ShapeDtypeStruct((_N, d), jnp.float32)
        out_specs = pl.BlockSpec((_MB, d), lambda i: (i, 0))
    return pl.pallas_call(
        body, grid=(_G,),
        in_specs=[pl.BlockSpec((C, _MB, _W), lambda i: (0, i, 0)),
                  pl.BlockSpec((_MB, 128), lambda i: (i, 0))],
        out_specs=out_specs, out_shape=out_shape)(part, norm)


def _scale(x, norm):
    d = x.shape[1]
    C = d // _W

    def body(x_ref, n_ref, y_ref):
        y = x_ref[...] * n_ref[...][:, 0:1]
        for c in range(C):
            y_ref[c] = y[:, c * _W:(c + 1) * _W]

    return pl.pallas_call(
        body, grid=(_G,),
        in_specs=[pl.BlockSpec((_MB, d), lambda i: (i, 0)),
                  pl.BlockSpec((_MB, 128), lambda i: (i, 0))],
        out_specs=pl.BlockSpec((C, _MB, _W), lambda i: (0, i, 0)),
        out_shape=jax.ShapeDtypeStruct((C, _NS, _W), jnp.float32))(x, norm)


def _norm_from_deg(deg_part):
    def body(p_ref, o_ref):
        d = p_ref[0][:, 0:1] + p_ref[1][:, 0:1]
        n = lax.rsqrt(jnp.maximum(d, 1.0))
        o_ref[...] = jnp.broadcast_to(n, (_MB, 128))

    return pl.pallas_call(
        body, grid=(_G,),
        in_specs=[pl.BlockSpec((2, _MB, 16), lambda i: (0, i, 0))],
        out_specs=pl.BlockSpec((_MB, 128), lambda i: (i, 0)),
        out_shape=jax.ShapeDtypeStruct((_N, 128), jnp.float32))(deg_part)


def _attention_weights(fs, al_v, ar_v, oh):
    rl = fs[0] * al_v
    ra = [f * ar_v for f in fs]
    weights = []
    for hh in range(_H):
        sl = slice(hh * oh, (hh + 1) * oh)
        a_l = jnp.sum(rl[:, sl], axis=1, keepdims=True)
        logits = [a_l + jnp.sum(r[:, sl], axis=1, keepdims=True) for r in ra]
        logits = [jnp.where(t >= 0, t, _NEG * t) for t in logits]
        mx = jnp.maximum(jnp.maximum(logits[0], logits[1]),
                         jnp.maximum(logits[2], logits[3]))
        es = [jnp.exp(t - mx) for t in logits]
        inv = 1.0 / (es[0] + es[1] + es[2] + es[3])
        weights.append([e * inv for e in es])
    return weights


def _combine(fs, lin, al_v, ar_v):
    d = lin.shape[1]
    oh = d // _H

    def body(f0, f1, f2, f3, l_ref, al_ref, ar_ref, o_ref, s_ref, q_ref):
        i = pl.program_id(0)
        fs_v = [f0[...], f1[...], f2[...], f3[...]]
        lin_v = l_ref[...]
        wts = _attention_weights(fs_v, al_ref[...], ar_ref[...], oh)
        cols = []
        for hh in range(_H):
            sl = slice(hh * oh, (hh + 1) * oh)
            acc = lin_v[:, sl]
            for k in range(4):
                acc = acc + fs_v[k][:, sl] * wts[hh][k]
            cols.append(acc)
        out = jnp.concatenate(cols, axis=1)
        o_ref[...] = out

        @pl.when(i == 0)
        def _():
            s_ref[...] = jnp.zeros_like(s_ref)
            q_ref[...] = jnp.zeros_like(q_ref)

        s_ref[...] += jnp.sum(out, axis=0, keepdims=True)
        q_ref[...] += jnp.sum(out * out, axis=0, keepdims=True)

    blk = pl.BlockSpec((_MB, d), lambda i: (i, 0))
    vec = pl.BlockSpec((1, d), lambda i: (0, 0))
    return pl.pallas_call(
        body, grid=(_G,),
        in_specs=[blk, blk, blk, blk, blk, vec, vec],
        out_specs=(blk, vec, vec),
        out_shape=(jax.ShapeDtypeStruct((_N, d), jnp.float32),
                   jax.ShapeDtypeStruct((1, d), jnp.float32),
                   jax.ShapeDtypeStruct((1, d), jnp.float32)),
    )(*fs, lin, al_v, ar_v)


def _combine_last(fs, lin, al_v, ar_v, bias):
    d = lin.shape[1]
    oh = d // _H

    def body(f0, f1, f2, f3, l_ref, al_ref, ar_ref, b_ref, o_ref):
        fs_v = [f0[...], f1[...], f2[...], f3[...]]
        lin_v = l_ref[...]
        wts = _attention_weights(fs_v, al_ref[...], ar_ref[...], oh)
        total = None
        for hh in range(_H):
            sl = slice(hh * oh, (hh + 1) * oh)
            acc = lin_v[:, sl]
            for k in range(4):
                acc = acc + fs_v[k][:, sl] * wts[hh][k]
            total = acc if total is None else total + acc
        o_ref[...] = total * (1.0 / _H) + b_ref[...]

    blk = pl.BlockSpec((_MB, d), lambda i: (i, 0))
    vec = pl.BlockSpec((1, d), lambda i: (0, 0))
    return pl.pallas_call(
        body, grid=(_G,),
        in_specs=[blk, blk, blk, blk, blk, vec, vec,
                  pl.BlockSpec((1, oh), lambda i: (0, 0))],
        out_specs=pl.BlockSpec((_MB, oh), lambda i: (i, 0)),
        out_shape=jax.ShapeDtypeStruct((_N, oh), jnp.float32),
    )(*fs, lin, al_v, ar_v, bias)


def _bn_relu(x, sums, sumsq, g, b):
    d = x.shape[1]

    def body(x_ref, s_ref, q_ref, g_ref, b_ref, o_ref):
        mu = s_ref[...] * (1.0 / _N)
        var = q_ref[...] * (1.0 / _N) - mu * mu
        rstd = lax.rsqrt(var + 1e-5)
        y = (x_ref[...] - mu) * (rstd * g_ref[...]) + b_ref[...]
        o_ref[...] = jnp.maximum(y, 0.0)

    vec = pl.BlockSpec((1, d), lambda i: (0, 0))
    return pl.pallas_call(
        body, grid=(_G,),
        in_specs=[pl.BlockSpec((_MB, d), lambda i: (i, 0)), vec, vec, vec, vec],
        out_specs=pl.BlockSpec((_MB, d), lambda i: (i, 0)),
        out_shape=jax.ShapeDtypeStruct((_N, d), jnp.float32),
    )(x, sums, sumsq, g, b)


def kernel(feat, edge_index, Wfc0, al0, ar0, Wlin0, g0, b0,
           Wfc1, al1, ar1, Wlin1, g1, b1, Wfc2, al2, ar2, Wlin2, bias_last):
    src = edge_index[0]
    dst = edge_index[1]
    pad = _EP - _E
    src_p = jnp.concatenate([src, jnp.zeros((pad,), jnp.int32)])
    dst_p = jnp.concatenate([dst, jnp.full((pad,), _N, jnp.int32)])
    dst3 = dst_p.reshape(32, _NSUB, 128)
    dst3h = dst_p.reshape(16, _NSUB2, 128)
    zeros_w = jnp.zeros((_RT, _W), jnp.float32)
    zeros_16 = jnp.zeros((_RT, 16), jnp.float32)
    ones_16 = jnp.ones((128, 16), jnp.float32)

    deg_part = _make_sc_degree()(dst3, ones_16, zeros_16)
    norm = _norm_from_deg(deg_part)

    def propagate(y):
        hop = _make_sc_hop(y.shape[0])
        return hop(y, src_p, dst3h, zeros_w)

    hs = [feat]
    y = _scale(feat, norm)
    for k in range(3):
        part = propagate(y)
        if k < 2:
            h_k, y = _hop_post(part, norm, True)
        else:
            h_k = _hop_post(part, norm, False)
        hs.append(h_k)
    fs = [_mm(h_k, Wfc0) for h_k in hs]
    lin = _mm(feat, Wlin0)
    out, s, q = _combine(fs, lin, al0.reshape(1, -1), ar0.reshape(1, -1))
    h = _bn_relu(out, s, q, g0.reshape(1, -1), b0.reshape(1, -1))

    f0, y = _mm(h, Wfc1, norm)
    lin = _mm(h, Wlin1)
    fs = [f0]
    for k in range(3):
        part = propagate(y)
        if k < 2:
            f_k, y = _hop_post(part, norm, True)
        else:
            f_k = _hop_post(part, norm, False)
        fs.append(f_k)
    out, s, q = _combine(fs, lin, al1.reshape(1, -1), ar1.reshape(1, -1))
    h = _bn_relu(out, s, q, g1.reshape(1, -1), b1.reshape(1, -1))

    f0, y = _mm(h, Wfc2, norm)
    lin = _mm(h, Wlin2)
    fs = [f0]
    for k in range(3):
        part = propagate(y)
        if k < 2:
            f_k, y = _hop_post(part, norm, True)
        else:
            f_k = _hop_post(part, norm, False)
        fs.append(f_k)
    return _combine_last(fs, lin, al2.reshape(1, -1), ar2.reshape(1, -1),
                         bias_last.reshape(1, -1))
